# baseline (speedup 1.0000x reference)
import functools

import jax
import jax.numpy as jnp
from jax import lax
from jax.experimental import pallas as pl
from jax.experimental.pallas import tpu as pltpu
from jax.experimental.pallas import tpu_sc as plsc

NUM_CLASSES = 1000
N_CTX = 16
CTX_DIM = 768
SEQ_LEN = 77
SUF_LEN = SEQ_LEN - 1 - N_CTX
LT = CTX_DIM // 128
U_CTX = NUM_CLASSES * (N_CTX // 8) * LT * 8
U_SUF = SUF_LEN * (NUM_CLASSES // 8) * LT * 8
U_OUT = SEQ_LEN * 128 * LT * 8
SLAB = 128 * LT * 8

try:
    _info = plsc.get_sparse_core_info()
    _NC, _NS = _info.num_cores, _info.num_subcores
except Exception:
    _NC, _NS = 2, 16
_NW = _NC * _NS
BPW = 1024 // _NW
UPW = (BPW // 8) * LT * 8
NBUF = 4


@jax.jit
def _gather_prompts(lab, a_ctx, a_pre, a_suf):
    mesh = plsc.VectorSubcoreMesh(core_axis_name="c", subcore_axis_name="s")

    @functools.partial(
        pl.kernel,
        mesh=mesh,
        out_type=jax.ShapeDtypeStruct((U_OUT, 128), jnp.float32),
        compiler_params=pltpu.CompilerParams(needs_layout_passes=False),
        scratch_types=[
            pltpu.VMEM((BPW,), jnp.int32),
            pltpu.VMEM((NBUF, 2, 96), jnp.int32),
        ] + [pltpu.VMEM((UPW, 128), jnp.float32)] * NBUF
          + [pltpu.SemaphoreType.DMA] * (2 * NBUF),
    )
    def body(lab_hbm, ctx_hbm, pre_hbm, suf_hbm, out_hbm, labv, jvb,
             buf0, buf1, buf2, buf3,
             gsem0, gsem1, gsem2, gsem3, wsem0, wsem1, wsem2, wsem3):
        wid = lax.axis_index("s") * _NC + lax.axis_index("c")
        bufs = (buf0, buf1, buf2, buf3)
        gsems = (gsem0, gsem1, gsem2, gsem3)
        wsems = (wsem0, wsem1, wsem2, wsem3)

        pltpu.sync_copy(lab_hbm.at[pl.ds(wid * BPW, BPW)], labv)

        iota16 = lax.iota(jnp.int32, 16)
        lane8 = iota16 % 8
        lgrp = iota16 // 8

        def fill_jvb(s, k):
            s32 = jnp.asarray(s, jnp.int32)

            def each_vreg(fn):
                for h in range(2):
                    for kk in range(LT):
                        bvec = (2 * h + kk // 3) * 8 + lane8
                        lvec = lgrp + (2 * kk) % LT
                        jvb[k, h, pl.ds(16 * kk, 16)] = fn(bvec, lvec)

            @pl.when(s32 == 0)
            def _():
                each_vreg(lambda bvec, lvec: lvec)

            @pl.when((s32 >= 1) & (s32 <= N_CTX))
            def _():
                ctx_base = ((s32 - 1) // 8) * (LT * 8) + (s32 - 1) % 8

                def f(bvec, lvec):
                    c = plsc.load_gather(labv, [bvec])
                    return c * (2 * LT * 8) + lvec * 8 + ctx_base
                each_vreg(f)

            @pl.when(s32 > N_CTX)
            def _():
                suf_base = (s32 - 1 - N_CTX) * (NUM_CLASSES // 8) * (LT * 8)

                def f(bvec, lvec):
                    c = plsc.load_gather(labv, [bvec])
                    return (c // 8) * (LT * 8) + c % 8 + lvec * 8 + suf_base
                each_vreg(f)

        def issue_gathers(s, k):
            s32 = jnp.asarray(s, jnp.int32)

            def issue(tab):
                def _go():
                    for h in range(2):
                        pltpu.async_copy(tab.at[jvb.at[k, h]],
                                         bufs[k].at[pl.ds(96 * h, 96)],
                                         gsems[k])
                return _go
            pl.when(s32 == 0)(issue(pre_hbm))
            pl.when((s32 >= 1) & (s32 <= N_CTX))(issue(ctx_hbm))
            pl.when(s32 > N_CTX)(issue(suf_hbm))

        def fill_and_issue(s, k):
            fill_jvb(s, k)
            issue_gathers(s, k)

        def drain_gathers(k):
            for h in range(2):
                pltpu.make_async_copy(ctx_hbm.at[pl.ds(0, 96)],
                                      bufs[k].at[pl.ds(96 * h, 96)],
                                      gsems[k]).wait()

        def write_slab(s, k):
            pltpu.async_copy(bufs[k],
                             out_hbm.at[pl.ds(s * SLAB + wid * UPW, UPW)],
                             wsems[k])

        def drain_write(k):
            pltpu.make_async_copy(bufs[k], out_hbm.at[pl.ds(0, UPW)],
                                  wsems[k]).wait()

        for k in range(NBUF):
            fill_and_issue(k, k)

        def group(g, carry):
            s0 = NBUF * g
            for k in range(NBUF):
                s = s0 + k
                kp = (k - 1) % NBUF
                drain_gathers(k)

                @pl.when(s + NBUF < SEQ_LEN)
                def _():
                    fill_jvb(s + NBUF, k)
                write_slab(s, k)
                sp = s - 1

                @pl.when((sp >= 0) & (sp + NBUF < SEQ_LEN))
                def _():
                    drain_write(kp)
                    issue_gathers(sp + NBUF, kp)
            return carry

        lax.fori_loop(0, (SEQ_LEN - 1) // NBUF, group, 0)
        drain_gathers(0)
        write_slab(SEQ_LEN - 1, 0)
        for k in range(1, NBUF):
            drain_write(k)
        drain_write(0)

    return body(lab, a_ctx, a_pre, a_suf)


def kernel(label, cls_ctx, token_prefix, token_suffix):
    a_ctx = cls_ctx.reshape(NUM_CLASSES, 2, 8, LT, 128).transpose(
        0, 1, 3, 2, 4).reshape(U_CTX, 128)
    a_pre = token_prefix.reshape(LT, 128)
    a_suf = token_suffix.reshape(NUM_CLASSES // 8, 8, SUF_LEN, LT,
                                 128).transpose(2, 0, 3, 1, 4).reshape(
                                     U_SUF, 128)
    o = _gather_prompts(label.astype(jnp.int32), a_ctx, a_pre, a_suf)
    return o.reshape(SEQ_LEN, 128, LT, 8, 128).transpose(
        1, 3, 0, 2, 4).reshape(128 * 8, SEQ_LEN, CTX_DIM)


# speedup vs baseline: 1.2188x; 1.2188x over previous
import functools

import jax
import jax.numpy as jnp
from jax import lax
from jax.experimental import pallas as pl
from jax.experimental.pallas import tpu as pltpu
from jax.experimental.pallas import tpu_sc as plsc

NUM_CLASSES = 1000
N_CTX = 16
CTX_DIM = 768
SEQ_LEN = 77
SUF_LEN = SEQ_LEN - 1 - N_CTX
LT = CTX_DIM // 128
U_CTX = NUM_CLASSES * (N_CTX // 8) * LT * 8
U_SUF = SUF_LEN * (NUM_CLASSES // 8) * LT * 8
U_OUT = SEQ_LEN * 128 * LT * 8
SLAB = 128 * LT * 8

try:
    _info = plsc.get_sparse_core_info()
    _NC, _NS = _info.num_cores, _info.num_subcores
except Exception:
    _NC, _NS = 2, 16
_NW = _NC * _NS
BPW = 1024 // _NW
UPW = (BPW // 8) * LT * 8
NBUF = 4


@jax.jit
def _gather_prompts(lab, a_ctx, a_pre, a_suf):
    mesh = plsc.VectorSubcoreMesh(core_axis_name="c", subcore_axis_name="s")

    @functools.partial(
        pl.kernel,
        mesh=mesh,
        out_type=jax.ShapeDtypeStruct((U_OUT, 128), jnp.float32),
        compiler_params=pltpu.CompilerParams(needs_layout_passes=False),
        scratch_types=[
            pltpu.VMEM((BPW,), jnp.int32),
            pltpu.VMEM((NBUF, 2, 96), jnp.int32),
            pltpu.VMEM((LT, 128), jnp.float32),
            pltpu.VMEM((LT * 8, 128), jnp.float32),
        ] + [pltpu.VMEM((UPW, 128), jnp.float32)] * NBUF
          + [pltpu.SemaphoreType.DMA] * (2 * NBUF + 1),
    )
    def body(lab_hbm, ctx_hbm, pre_hbm, suf_hbm, out_hbm, labv, jvb,
             pre_v, pre_slab, buf0, buf1, buf2, buf3,
             gsem0, gsem1, gsem2, gsem3, wsem0, wsem1, wsem2, wsem3,
             psem):
        wid = lax.axis_index("s") * _NC + lax.axis_index("c")
        bufs = (buf0, buf1, buf2, buf3)
        gsems = (gsem0, gsem1, gsem2, gsem3)
        wsems = (wsem0, wsem1, wsem2, wsem3)

        pltpu.sync_copy(lab_hbm.at[pl.ds(wid * BPW, BPW)], labv)

        iota16 = lax.iota(jnp.int32, 16)
        lane8 = iota16 % 8
        lgrp = iota16 // 8

        def fill_jvb(s, k):
            s32 = jnp.asarray(s, jnp.int32)

            def each_vreg(fn):
                for h in range(2):
                    for kk in range(LT):
                        bvec = (2 * h + kk // 3) * 8 + lane8
                        lvec = lgrp + (2 * kk) % LT
                        jvb[k, h, pl.ds(16 * kk, 16)] = fn(bvec, lvec)

            @pl.when(s32 <= N_CTX)
            def _():
                ctx_base = ((s32 - 1) // 8) * (LT * 8) + (s32 - 1) % 8

                def f(bvec, lvec):
                    c = plsc.load_gather(labv, [bvec])
                    return c * (2 * LT * 8) + lvec * 8 + ctx_base
                each_vreg(f)

            @pl.when(s32 > N_CTX)
            def _():
                suf_base = (s32 - 1 - N_CTX) * (NUM_CLASSES // 8) * (LT * 8)

                def f(bvec, lvec):
                    c = plsc.load_gather(labv, [bvec])
                    return (c // 8) * (LT * 8) + c % 8 + lvec * 8 + suf_base
                each_vreg(f)

        def issue_gathers(s, k):
            s32 = jnp.asarray(s, jnp.int32)

            def issue(tab):
                def _go():
                    for h in range(2):
                        pltpu.async_copy(tab.at[jvb.at[k, h]],
                                         bufs[k].at[pl.ds(96 * h, 96)],
                                         gsems[k])
                return _go
            pl.when(s32 <= N_CTX)(issue(ctx_hbm))
            pl.when(s32 > N_CTX)(issue(suf_hbm))

        def fill_and_issue(s, k):
            fill_jvb(s, k)
            issue_gathers(s, k)

        def drain_gathers(k):
            for h in range(2):
                pltpu.make_async_copy(ctx_hbm.at[pl.ds(0, 96)],
                                      bufs[k].at[pl.ds(96 * h, 96)],
                                      gsems[k]).wait()

        def write_slab(s, k):
            pltpu.async_copy(bufs[k],
                             out_hbm.at[pl.ds(s * SLAB + wid * UPW, UPW)],
                             wsems[k])

        def drain_write(k):
            pltpu.make_async_copy(bufs[k], out_hbm.at[pl.ds(0, UPW)],
                                  wsems[k]).wait()

        for k in range(NBUF):
            fill_and_issue(k + 1, k)
        pltpu.sync_copy(pre_hbm, pre_v)
        for l in range(LT):
            for j in range(8):
                v = pre_v[l, pl.ds(16 * j, 16)]
                for r in range(8):
                    pre_slab[8 * l + r, pl.ds(16 * j, 16)] = v
        for t in range(4):
            pltpu.async_copy(
                pre_slab, out_hbm.at[pl.ds(wid * UPW + 48 * t, 48)], psem)

        def group(g, carry):
            for k in range(NBUF):
                s = NBUF * g + k + 1
                kp = (k - 1) % NBUF
                drain_gathers(k)

                @pl.when(s + NBUF < SEQ_LEN)
                def _():
                    fill_jvb(s + NBUF, k)
                write_slab(s, k)
                sp = s - 1

                @pl.when((sp >= 1) & (sp + NBUF < SEQ_LEN))
                def _():
                    drain_write(kp)
                    issue_gathers(sp + NBUF, kp)
            return carry

        lax.fori_loop(0, (SEQ_LEN - 1) // NBUF, group, 0)
        for k in range(NBUF):
            drain_write(k)
        for t in range(4):
            pltpu.make_async_copy(pre_slab, out_hbm.at[pl.ds(0, 48)],
                                  psem).wait()

    return body(lab, a_ctx, a_pre, a_suf)


def kernel(label, cls_ctx, token_prefix, token_suffix):
    a_ctx = cls_ctx.reshape(NUM_CLASSES, 2, 8, LT, 128).transpose(
        0, 1, 3, 2, 4).reshape(U_CTX, 128)
    a_pre = token_prefix.reshape(LT, 128)
    a_suf = token_suffix.reshape(NUM_CLASSES // 8, 8, SUF_LEN, LT,
                                 128).transpose(2, 0, 3, 1, 4).reshape(
                                     U_SUF, 128)
    o = _gather_prompts(label.astype(jnp.int32), a_ctx, a_pre, a_suf)
    return o.reshape(SEQ_LEN, 128, LT, 8, 128).transpose(
        1, 3, 0, 2, 4).reshape(128 * 8, SEQ_LEN, CTX_DIM)


# speedup vs baseline: 1.2293x; 1.0087x over previous
import functools

import jax
import jax.numpy as jnp
from jax import lax
from jax.experimental import pallas as pl
from jax.experimental.pallas import tpu as pltpu
from jax.experimental.pallas import tpu_sc as plsc

NUM_CLASSES = 1000
N_CTX = 16
CTX_DIM = 768
SEQ_LEN = 77
SUF_LEN = SEQ_LEN - 1 - N_CTX
LT = CTX_DIM // 128
U_CTX = NUM_CLASSES * (N_CTX // 8) * LT * 8
U_SUF = SUF_LEN * (NUM_CLASSES // 8) * LT * 8
U_OUT = SEQ_LEN * 128 * LT * 8
SLAB = 128 * LT * 8

try:
    _info = plsc.get_sparse_core_info()
    _NC, _NS = _info.num_cores, _info.num_subcores
except Exception:
    _NC, _NS = 2, 16
_NW = _NC * _NS
BPW = 1024 // _NW
UPW = (BPW // 8) * LT * 8
NBUF = 4


@jax.jit
def _gather_prompts(lab, a_ctx, a_pre, a_suf):
    mesh = plsc.VectorSubcoreMesh(core_axis_name="c", subcore_axis_name="s")

    @functools.partial(
        pl.kernel,
        mesh=mesh,
        out_type=jax.ShapeDtypeStruct((U_OUT, 128), jnp.float32),
        compiler_params=pltpu.CompilerParams(needs_layout_passes=False),
        scratch_types=[
            pltpu.VMEM((BPW,), jnp.int32),
            pltpu.VMEM((NBUF, 2, 96), jnp.int32),
            pltpu.VMEM((LT, 128), jnp.float32),
            pltpu.VMEM((LT * 8, 128), jnp.float32),
        ] + [pltpu.VMEM((UPW, 128), jnp.float32)] * NBUF
          + [pltpu.SemaphoreType.DMA] * (2 * NBUF + 1),
    )
    def body(lab_hbm, ctx_hbm, pre_hbm, suf_hbm, out_hbm, labv, jvb,
             pre_v, pre_slab, buf0, buf1, buf2, buf3,
             gsem0, gsem1, gsem2, gsem3, wsem0, wsem1, wsem2, wsem3,
             psem):
        wid = lax.axis_index("s") * _NC + lax.axis_index("c")
        bufs = (buf0, buf1, buf2, buf3)
        gsems = (gsem0, gsem1, gsem2, gsem3)
        wsems = (wsem0, wsem1, wsem2, wsem3)

        pltpu.sync_copy(lab_hbm.at[pl.ds(wid * BPW, BPW)], labv)

        iota16 = lax.iota(jnp.int32, 16)
        lane8 = iota16 % 8
        lgrp = iota16 // 8

        def precompute_terms():
            ctx_t, suf_t = [], []
            for w in range(4):
                c = plsc.load_gather(labv, [w * 8 + lane8])
                ctx_t.append(c * (2 * LT * 8) + lgrp * 8)
                suf_t.append((c // 8) * (LT * 8) + c % 8 + lgrp * 8)
            return tuple(ctx_t) + tuple(suf_t)

        def fill_jvb(s, k, terms):
            s32 = jnp.asarray(s, jnp.int32)

            def each_vreg(base, toff):
                for h in range(2):
                    for kk in range(LT):
                        w = 2 * h + kk // 3
                        jvb[k, h, pl.ds(16 * kk, 16)] = (
                            terms[toff + w] + (base + 8 * ((2 * kk) % LT)))

            @pl.when(s32 <= N_CTX)
            def _():
                each_vreg(((s32 - 1) // 8) * (LT * 8) + (s32 - 1) % 8, 0)

            @pl.when(s32 > N_CTX)
            def _():
                each_vreg((s32 - 1 - N_CTX) * (NUM_CLASSES // 8) * (LT * 8),
                          4)

        def issue_gathers(s, k):
            s32 = jnp.asarray(s, jnp.int32)

            def issue(tab):
                def _go():
                    for h in range(2):
                        pltpu.async_copy(tab.at[jvb.at[k, h]],
                                         bufs[k].at[pl.ds(96 * h, 96)],
                                         gsems[k])
                return _go
            pl.when(s32 <= N_CTX)(issue(ctx_hbm))
            pl.when(s32 > N_CTX)(issue(suf_hbm))

        def fill_and_issue(s, k, terms):
            fill_jvb(s, k, terms)
            issue_gathers(s, k)

        def drain_gathers(k):
            for h in range(2):
                pltpu.make_async_copy(ctx_hbm.at[pl.ds(0, 96)],
                                      bufs[k].at[pl.ds(96 * h, 96)],
                                      gsems[k]).wait()

        def write_slab(s, k):
            pltpu.async_copy(bufs[k],
                             out_hbm.at[pl.ds(s * SLAB + wid * UPW, UPW)],
                             wsems[k])

        def drain_write(k):
            pltpu.make_async_copy(bufs[k], out_hbm.at[pl.ds(0, UPW)],
                                  wsems[k]).wait()

        terms0 = precompute_terms()
        for k in range(NBUF):
            fill_and_issue(k + 1, k, terms0)
        pltpu.sync_copy(pre_hbm, pre_v)
        for l in range(LT):
            for j in range(8):
                v = pre_v[l, pl.ds(16 * j, 16)]
                for r in range(8):
                    pre_slab[8 * l + r, pl.ds(16 * j, 16)] = v
        for t in range(4):
            pltpu.async_copy(
                pre_slab, out_hbm.at[pl.ds(wid * UPW + 48 * t, 48)], psem)

        def group(g, terms):
            for k in range(NBUF):
                s = NBUF * g + k + 1
                kp = (k - 1) % NBUF
                drain_gathers(k)

                @pl.when(s + NBUF < SEQ_LEN)
                def _():
                    fill_jvb(s + NBUF, k, terms)
                write_slab(s, k)
                sp = s - 1

                @pl.when((sp >= 1) & (sp + NBUF < SEQ_LEN))
                def _():
                    drain_write(kp)
                    issue_gathers(sp + NBUF, kp)
            return terms

        lax.fori_loop(0, (SEQ_LEN - 1) // NBUF, group, terms0)
        for k in range(NBUF):
            drain_write(k)
        for t in range(4):
            pltpu.make_async_copy(pre_slab, out_hbm.at[pl.ds(0, 48)],
                                  psem).wait()

    return body(lab, a_ctx, a_pre, a_suf)


def kernel(label, cls_ctx, token_prefix, token_suffix):
    a_ctx = cls_ctx.reshape(NUM_CLASSES, 2, 8, LT, 128).transpose(
        0, 1, 3, 2, 4).reshape(U_CTX, 128)
    a_pre = token_prefix.reshape(LT, 128)
    a_suf = token_suffix.reshape(NUM_CLASSES // 8, 8, SUF_LEN, LT,
                                 128).transpose(2, 0, 3, 1, 4).reshape(
                                     U_SUF, 128)
    o = _gather_prompts(label.astype(jnp.int32), a_ctx, a_pre, a_suf)
    return o.reshape(SEQ_LEN, 128, LT, 8, 128).transpose(
        1, 3, 0, 2, 4).reshape(128 * 8, SEQ_LEN, CTX_DIM)


# speedup vs baseline: 1.2372x; 1.0064x over previous
import functools

import jax
import jax.numpy as jnp
from jax import lax
from jax.experimental import pallas as pl
from jax.experimental.pallas import tpu as pltpu
from jax.experimental.pallas import tpu_sc as plsc

NUM_CLASSES = 1000
N_CTX = 16
CTX_DIM = 768
SEQ_LEN = 77
SUF_LEN = SEQ_LEN - 1 - N_CTX
LT = CTX_DIM // 128
U_CTX = NUM_CLASSES * (N_CTX // 8) * LT * 8
U_SUF = SUF_LEN * (NUM_CLASSES // 8) * LT * 8
U_OUT = SEQ_LEN * 128 * LT * 8
SLAB = 128 * LT * 8

try:
    _info = plsc.get_sparse_core_info()
    _NC, _NS = _info.num_cores, _info.num_subcores
except Exception:
    _NC, _NS = 2, 16
_NW = _NC * _NS
BPW = 1024 // _NW
UPW = (BPW // 8) * LT * 8
NBUF = 5


@jax.jit
def _gather_prompts(lab, a_ctx, a_pre, a_suf):
    mesh = plsc.VectorSubcoreMesh(core_axis_name="c", subcore_axis_name="s")

    @functools.partial(
        pl.kernel,
        mesh=mesh,
        out_type=jax.ShapeDtypeStruct((U_OUT, 128), jnp.float32),
        compiler_params=pltpu.CompilerParams(needs_layout_passes=False),
        scratch_types=[
            pltpu.VMEM((BPW,), jnp.int32),
            pltpu.VMEM((NBUF, 2, 96), jnp.int32),
        ] + [pltpu.VMEM((UPW, 128), jnp.float32)] * NBUF
          + [pltpu.SemaphoreType.DMA] * (2 * NBUF + 1),
    )
    def body(lab_hbm, ctx_hbm, pre_hbm, suf_hbm, out_hbm, labv, jvb,
             buf0, buf1, buf2, buf3, buf4,
             gsem0, gsem1, gsem2, gsem3, gsem4,
             wsem0, wsem1, wsem2, wsem3, wsem4, psem):
        wid = lax.axis_index("s") * _NC + lax.axis_index("c")
        bufs = (buf0, buf1, buf2, buf3, buf4)
        gsems = (gsem0, gsem1, gsem2, gsem3, gsem4)
        wsems = (wsem0, wsem1, wsem2, wsem3, wsem4)

        pltpu.sync_copy(lab_hbm.at[pl.ds(wid * BPW, BPW)], labv)

        iota16 = lax.iota(jnp.int32, 16)
        lane8 = iota16 % 8
        lgrp = iota16 // 8

        def precompute_terms():
            ctx_t, suf_t = [], []
            for w in range(4):
                c = plsc.load_gather(labv, [w * 8 + lane8])
                ctx_t.append(c * (2 * LT * 8) + lgrp * 8)
                suf_t.append((c // 8) * (LT * 8) + c % 8 + lgrp * 8)
            return tuple(ctx_t) + tuple(suf_t)

        def fill_jvb(s, k, terms):
            s32 = jnp.asarray(s, jnp.int32)

            def each_vreg(base, toff):
                for h in range(2):
                    for kk in range(LT):
                        w = 2 * h + kk // 3
                        jvb[k, h, pl.ds(16 * kk, 16)] = (
                            terms[toff + w] + (base + 8 * ((2 * kk) % LT)))

            @pl.when(s32 <= N_CTX)
            def _():
                each_vreg(((s32 - 1) // 8) * (LT * 8) + (s32 - 1) % 8, 0)

            @pl.when(s32 > N_CTX)
            def _():
                each_vreg((s32 - 1 - N_CTX) * (NUM_CLASSES // 8) * (LT * 8),
                          4)

        def issue_gathers(s, k):
            s32 = jnp.asarray(s, jnp.int32)

            def issue(tab):
                def _go():
                    for h in range(2):
                        pltpu.async_copy(tab.at[jvb.at[k, h]],
                                         bufs[k].at[pl.ds(96 * h, 96)],
                                         gsems[k])
                return _go
            pl.when(s32 <= N_CTX)(issue(ctx_hbm))
            pl.when(s32 > N_CTX)(issue(suf_hbm))

        def fill_and_issue(s, k, terms):
            fill_jvb(s, k, terms)
            issue_gathers(s, k)

        def drain_gathers(k):
            for h in range(2):
                pltpu.make_async_copy(ctx_hbm.at[pl.ds(0, 96)],
                                      bufs[k].at[pl.ds(96 * h, 96)],
                                      gsems[k]).wait()

        def write_slab(s, k):
            pltpu.async_copy(bufs[k],
                             out_hbm.at[pl.ds(s * SLAB + wid * UPW, UPW)],
                             wsems[k])

        def drain_write(k):
            pltpu.make_async_copy(bufs[k], out_hbm.at[pl.ds(0, UPW)],
                                  wsems[k]).wait()

        pltpu.sync_copy(pre_hbm, buf1.at[pl.ds(0, LT)])
        for l in range(LT):
            for j in range(8):
                v = buf1[l, pl.ds(16 * j, 16)]
                for r in range(8):
                    buf0[8 * l + r, pl.ds(16 * j, 16)] = v
        for t in range(4):
            pltpu.async_copy(
                buf0.at[pl.ds(0, 48)],
                out_hbm.at[pl.ds(wid * UPW + 48 * t, 48)], psem)
        for t in range(4):
            pltpu.make_async_copy(buf0.at[pl.ds(0, 48)],
                                  out_hbm.at[pl.ds(0, 48)], psem).wait()
        terms0 = precompute_terms()
        for k in range(NBUF):
            fill_and_issue(k + 1, k, terms0)

        def group(g, terms):
            for k in range(NBUF):
                s = NBUF * g + k + 1
                kp = (k - 1) % NBUF
                drain_gathers(k)

                @pl.when(s + NBUF < SEQ_LEN)
                def _():
                    fill_jvb(s + NBUF, k, terms)
                write_slab(s, k)
                sp = s - 1

                @pl.when((sp >= 1) & (sp + NBUF < SEQ_LEN))
                def _():
                    drain_write(kp)
                    issue_gathers(sp + NBUF, kp)
            return terms

        lax.fori_loop(0, (SEQ_LEN - 1) // NBUF, group, terms0)
        drain_gathers((SEQ_LEN - 2) % NBUF)
        write_slab(SEQ_LEN - 1, (SEQ_LEN - 2) % NBUF)
        for k in range(NBUF):
            drain_write(k)

    return body(lab, a_ctx, a_pre, a_suf)


def kernel(label, cls_ctx, token_prefix, token_suffix):
    a_ctx = cls_ctx.reshape(NUM_CLASSES, 2, 8, LT, 128).transpose(
        0, 1, 3, 2, 4).reshape(U_CTX, 128)
    a_pre = token_prefix.reshape(LT, 128)
    a_suf = token_suffix.reshape(NUM_CLASSES // 8, 8, SUF_LEN, LT,
                                 128).transpose(2, 0, 3, 1, 4).reshape(
                                     U_SUF, 128)
    o = _gather_prompts(label.astype(jnp.int32), a_ctx, a_pre, a_suf)
    return o.reshape(SEQ_LEN, 128, LT, 8, 128).transpose(
        1, 3, 0, 2, 4).reshape(128 * 8, SEQ_LEN, CTX_DIM)
